# baseline (speedup 1.0000x reference)
import functools

import jax
import jax.numpy as jnp
from jax import lax
from jax.experimental import pallas as pl
from jax.experimental.pallas import tpu as pltpu
from jax.experimental.pallas import tpu_sc as plsc

_K = 1024
_D = 64
_B = 64
_HW = 576
_N = _B * _HW

_NC = 2
_NS = 16
_NW = _NC * _NS
_BPW = _N // _NW
_CHUNK = 128
_NCHUNK = _BPW // _CHUNK


def _dist_argmin_body(z_ref, e_ref, idx_ref):
    z = z_ref[0]
    e = e_ref[...]
    a2 = jnp.sum(z * z, axis=1, keepdims=True)
    b2 = jnp.sum(e * e, axis=1)[None, :]
    dot = lax.dot_general(z, e, (((1,), (1,)), ((), ())),
                          preferred_element_type=jnp.float32)
    sq = a2 + b2 - 2.0 * dot
    d = jnp.sqrt(jnp.maximum(sq, 0.0))
    dmin = jnp.min(d, axis=1, keepdims=True)
    cols = lax.broadcasted_iota(jnp.int32, (_HW, _K), 1)
    cand = jnp.where(d == dmin, cols, _K)
    idx_ref[0, 0, :] = jnp.min(cand, axis=1).astype(jnp.int32)


def _combine_body(z_ref, q_ref, out_ref, ls_ref):
    z = z_ref[0]
    q = q_ref[0]
    d = q - z
    out_ref[0] = jnp.transpose(z + d, (1, 0))

    @pl.when(pl.program_id(0) == 0)
    def _init():
        ls_ref[...] = jnp.zeros((1, 1), jnp.float32)

    ls_ref[...] += jnp.sum(d * d).reshape(1, 1)


def _sc_gather_body(table_hbm, idx_hbm, out_hbm, idx_v, rows_v, sem):
    wid = lax.axis_index("s") * _NC + lax.axis_index("c")
    base = wid * _BPW
    pltpu.sync_copy(idx_hbm.at[pl.ds(base, _BPW)], idx_v)
    copies = [
        pltpu.async_copy(
            table_hbm.at[idx_v.at[pl.ds(j * _CHUNK, _CHUNK)]],
            rows_v.at[pl.ds(j * _CHUNK, _CHUNK)],
            sem,
        )
        for j in range(_NCHUNK)
    ]
    for c in copies:
        c.wait()
    pltpu.sync_copy(rows_v, out_hbm.at[pl.ds(base, _BPW)])


def kernel(z, embeddings):
    z3 = z.reshape(_B, _HW, _D)

    idx = pl.pallas_call(
        _dist_argmin_body,
        grid=(_B,),
        in_specs=[
            pl.BlockSpec((1, _HW, _D), lambda i: (i, 0, 0)),
            pl.BlockSpec((_K, _D), lambda i: (0, 0)),
        ],
        out_specs=pl.BlockSpec((1, 1, _HW), lambda i: (i, 0, 0)),
        out_shape=jax.ShapeDtypeStruct((_B, 1, _HW), jnp.int32),
    )(z3, embeddings)

    sc_gather = pl.kernel(
        _sc_gather_body,
        mesh=plsc.VectorSubcoreMesh(core_axis_name="c", subcore_axis_name="s"),
        out_type=jax.ShapeDtypeStruct((_N, _D), jnp.float32),
        scratch_types=[
            pltpu.VMEM((_BPW,), jnp.int32),
            pltpu.VMEM((_BPW, _D), jnp.float32),
            pltpu.SemaphoreType.DMA,
        ],
        compiler_params=pltpu.CompilerParams(use_tc_tiling_on_sc=False),
    )
    q_flat = sc_gather(embeddings, idx.reshape(_N))

    out3, lsum = pl.pallas_call(
        _combine_body,
        grid=(_B,),
        in_specs=[
            pl.BlockSpec((1, _HW, _D), lambda i: (i, 0, 0)),
            pl.BlockSpec((1, _HW, _D), lambda i: (i, 0, 0)),
        ],
        out_specs=[
            pl.BlockSpec((1, _D, _HW), lambda i: (i, 0, 0)),
            pl.BlockSpec((1, 1), lambda i: (0, 0)),
        ],
        out_shape=[
            jax.ShapeDtypeStruct((_B, _D, _HW), jnp.float32),
            jax.ShapeDtypeStruct((1, 1), jnp.float32),
        ],
    )(z3, q_flat.reshape(_B, _HW, _D))

    quantized = out3.reshape(_B, _D, 24, 24)
    quantized_indices = idx.reshape(_B, 1, 24, 24)
    m = lsum[0, 0] / jnp.float32(_N * _D)
    loss = m * jnp.float32(0.25) + m * jnp.float32(1.0)
    return (quantized, quantized_indices, loss)


# speedup vs baseline: 1.0478x; 1.0478x over previous
import jax
import jax.numpy as jnp
from jax import lax
from jax.experimental import pallas as pl
from jax.experimental.pallas import tpu as pltpu
from jax.experimental.pallas import tpu_sc as plsc

_K = 1024
_D = 64
_B = 64
_H = 24
_W = 24
_HW = _H * _W
_N = _B * _HW

_NC = 2
_NS = 16
_NW = _NC * _NS

_HB = _B // 2
_NH = _HB * _HW
_BPW = _NH // _NW
_CHUNK = 96
_NCHUNK = _BPW // _CHUNK

_IMG_PER_PROG = 2
_T = _IMG_PER_PROG * _HW
_G1 = _HB // _IMG_PER_PROG


def _dist_argmin_body(z_ref, e_ref, idx_ref, b2_ref):
    @pl.when(pl.program_id(0) == 0)
    def _b2():
        e0 = e_ref[...]
        b2_ref[...] = jnp.sum(e0 * e0, axis=1)[None, :]

    z = z_ref[...].reshape(_T, _D)
    e = e_ref[...]
    a2 = jnp.sum(z * z, axis=1, keepdims=True)
    dot = lax.dot_general(z, e, (((1,), (1,)), ((), ())),
                          preferred_element_type=jnp.float32)
    tp = jnp.maximum(a2 + b2_ref[...] - 2.0 * dot, 0.0)
    d = jnp.sqrt(tp)
    dc = [lax.slice(d, (0, c * 128), (_T, (c + 1) * 128)) for c in range(8)]
    acc = dc[0]
    for c in range(1, 8):
        acc = jnp.minimum(acc, dc[c])
    dmin = jnp.min(acc, axis=1, keepdims=True)
    dminb = jnp.broadcast_to(dmin, (_T, 128))
    two = jnp.float32(2.0)
    best = jnp.full((_T, 128), two, jnp.float32)
    for c in range(8):
        pc = lax.bitcast_convert_type(
            jnp.int32(0x3F800000 + c * 128)
            + lax.broadcasted_iota(jnp.int32, (1, 128), 1), jnp.float32)
        best = jnp.minimum(best, jnp.where(dc[c] == dminb, pc, two))
    pmin = jnp.min(best, axis=1, keepdims=True)
    idx1 = lax.bitcast_convert_type(pmin, jnp.int32) - jnp.int32(0x3F800000)
    idx_ref[...] = idx1.reshape(_IMG_PER_PROG, _NCHUNK, _CHUNK)


def _combine_body(z_ref, q_ref, out_ref, ls_ref):
    z = z_ref[...].reshape(_HW, _D)
    q = q_ref[...]
    d = q - z
    out_ref[...] = jnp.transpose(z + d, (1, 0)).reshape(1, _D, _H, _W)

    @pl.when(pl.program_id(0) == 0)
    def _init():
        ls_ref[...] = jnp.zeros((1, 1), jnp.float32)

    ls_ref[...] += jnp.sum(d * d).reshape(1, 1)


def _sc_gather_body(table_hbm, idx_hbm, out_hbm, idx_v, rows_v, sem):
    wid = lax.axis_index("s") * _NC + lax.axis_index("c")
    pltpu.sync_copy(idx_hbm.at[wid], idx_v)
    copies = [
        pltpu.async_copy(
            table_hbm.at[idx_v.at[j]],
            rows_v.at[pl.ds(j * _CHUNK, _CHUNK)],
            sem,
        )
        for j in range(_NCHUNK)
    ]
    for c in copies:
        c.wait()
    pltpu.sync_copy(rows_v, out_hbm.at[pl.ds(wid * _BPW, _BPW)])


def _half_pipeline(z, embeddings, sc_gather, half):
    base = half * _G1

    idx = pl.pallas_call(
        _dist_argmin_body,
        grid=(_G1,),
        in_specs=[
            pl.BlockSpec((_IMG_PER_PROG, _H, _W, _D),
                         lambda i, b=base: (i + b, 0, 0, 0)),
            pl.BlockSpec((_K, _D), lambda i: (0, 0)),
        ],
        out_specs=pl.BlockSpec((_IMG_PER_PROG, _NCHUNK, _CHUNK),
                               lambda i: (i, 0, 0)),
        out_shape=jax.ShapeDtypeStruct((_NW, _NCHUNK, _CHUNK), jnp.int32),
        scratch_shapes=[pltpu.VMEM((1, _K), jnp.float32)],
    )(z, embeddings)

    q_half = sc_gather(embeddings, idx)

    out4, lsum = pl.pallas_call(
        _combine_body,
        grid=(_HB,),
        in_specs=[
            pl.BlockSpec((1, _H, _W, _D),
                         lambda i, b=half * _HB: (i + b, 0, 0, 0)),
            pl.BlockSpec((_HW, _D), lambda i: (i, 0)),
        ],
        out_specs=[
            pl.BlockSpec((1, _D, _H, _W), lambda i: (i, 0, 0, 0)),
            pl.BlockSpec((1, 1), lambda i: (0, 0)),
        ],
        out_shape=[
            jax.ShapeDtypeStruct((_HB, _D, _H, _W), jnp.float32),
            jax.ShapeDtypeStruct((1, 1), jnp.float32),
        ],
    )(z, q_half)
    return idx, out4, lsum


def kernel(z, embeddings):
    sc_gather = pl.kernel(
        _sc_gather_body,
        mesh=plsc.VectorSubcoreMesh(core_axis_name="c", subcore_axis_name="s"),
        out_type=jax.ShapeDtypeStruct((_NH, _D), jnp.float32),
        scratch_types=[
            pltpu.VMEM((_NCHUNK, _CHUNK), jnp.int32),
            pltpu.VMEM((_BPW, _D), jnp.float32),
            pltpu.SemaphoreType.DMA,
        ],
        compiler_params=pltpu.CompilerParams(use_tc_tiling_on_sc=False),
    )

    idx_a, out_a, ls_a = _half_pipeline(z, embeddings, sc_gather, 0)
    idx_b, out_b, ls_b = _half_pipeline(z, embeddings, sc_gather, 1)

    quantized = jnp.concatenate([out_a, out_b], axis=0)
    quantized_indices = jnp.concatenate(
        [idx_a.reshape(_HB, 1, _H, _W), idx_b.reshape(_HB, 1, _H, _W)], axis=0)
    m = (ls_a[0, 0] + ls_b[0, 0]) / jnp.float32(_N * _D)
    loss = m * jnp.float32(0.25) + m * jnp.float32(1.0)
    return (quantized, quantized_indices, loss)


# speedup vs baseline: 1.1314x; 1.0797x over previous
import jax
import jax.numpy as jnp
from jax import lax
from jax.experimental import pallas as pl
from jax.experimental.pallas import tpu as pltpu
from jax.experimental.pallas import tpu_sc as plsc

_K = 1024
_D = 64
_DP = 128
_B = 64
_H = 24
_W = 24
_HW = _H * _W
_N = _B * _HW

_NC = 2
_NS = 16
_NW = _NC * _NS
_BPW = _N // _NW
_CHUNK = 128
_NCHUNK = _BPW // _CHUNK

_IMG_PER_PROG = 2
_T = _IMG_PER_PROG * _HW
_G1 = _B // _IMG_PER_PROG


def _dist_argmin_body(z_ref, e_ref, idx_ref, b2_ref):
    @pl.when(pl.program_id(0) == 0)
    def _b2():
        e0 = e_ref[...]
        b2_ref[...] = jnp.sum(e0 * e0, axis=1)[None, :]

    z = z_ref[...].reshape(_T, _D)
    e = e_ref[...]
    a2 = jnp.sum(z * z, axis=1, keepdims=True)
    dot = lax.dot_general(z, e, (((1,), (1,)), ((), ())),
                          preferred_element_type=jnp.float32)
    tp = jnp.maximum(a2 + b2_ref[...] - 2.0 * dot, 0.0)
    d = jnp.sqrt(tp)
    dc = [lax.slice(d, (0, c * 128), (_T, (c + 1) * 128)) for c in range(8)]
    acc = dc[0]
    for c in range(1, 8):
        acc = jnp.minimum(acc, dc[c])
    dmin = jnp.min(acc, axis=1, keepdims=True)
    dminb = jnp.broadcast_to(dmin, (_T, 128))
    two = jnp.float32(2.0)
    best = jnp.full((_T, 128), two, jnp.float32)
    for c in range(8):
        pc = lax.bitcast_convert_type(
            jnp.int32(0x3F800000 + c * 128)
            + lax.broadcasted_iota(jnp.int32, (1, 128), 1), jnp.float32)
        best = jnp.minimum(best, jnp.where(dc[c] == dminb, pc, two))
    pmin = jnp.min(best, axis=1, keepdims=True)
    idx1 = lax.bitcast_convert_type(pmin, jnp.int32) - jnp.int32(0x3F800000)
    idx_ref[...] = idx1.reshape(1, _NCHUNK, _CHUNK)


def _combine_body(z_ref, q_ref, out_ref, ls_ref):
    z = z_ref[...].reshape(_HW, _D)
    q = q_ref[...][:, :_D]
    d = q - z
    out_ref[...] = jnp.transpose(z + d, (1, 0)).reshape(1, _D, _H, _W)

    @pl.when(pl.program_id(0) == 0)
    def _init():
        ls_ref[...] = jnp.zeros((1, 1), jnp.float32)

    ls_ref[...] += jnp.sum(d * d).reshape(1, 1)


def _sc_gather_body(table_hbm, idx_hbm, out_hbm, idx_v, buf0, buf1, sem):
    wid = lax.axis_index("s") * _NC + lax.axis_index("c")
    base = wid * _BPW
    pltpu.sync_copy(idx_hbm.at[wid], idx_v)
    bufs = [buf0, buf1]
    copies = []
    for j in range(_NCHUNK):
        copies.append(pltpu.async_copy(
            table_hbm.at[idx_v.at[j]], bufs[j % 2], sem))
        if j > 0:
            copies[j - 1].wait()
            pltpu.sync_copy(
                bufs[(j - 1) % 2],
                out_hbm.at[pl.ds(base + (j - 1) * _CHUNK, _CHUNK)])
    copies[_NCHUNK - 1].wait()
    pltpu.sync_copy(
        bufs[(_NCHUNK - 1) % 2],
        out_hbm.at[pl.ds(base + (_NCHUNK - 1) * _CHUNK, _CHUNK)])


def kernel(z, embeddings):
    emb_p = jnp.pad(embeddings, ((0, 0), (0, _DP - _D)))

    idx = pl.pallas_call(
        _dist_argmin_body,
        grid=(_G1,),
        in_specs=[
            pl.BlockSpec((_IMG_PER_PROG, _H, _W, _D), lambda i: (i, 0, 0, 0)),
            pl.BlockSpec((_K, _D), lambda i: (0, 0)),
        ],
        out_specs=pl.BlockSpec((1, _NCHUNK, _CHUNK), lambda i: (i, 0, 0)),
        out_shape=jax.ShapeDtypeStruct((_NW, _NCHUNK, _CHUNK), jnp.int32),
        scratch_shapes=[pltpu.VMEM((1, _K), jnp.float32)],
    )(z, embeddings)

    sc_gather = pl.kernel(
        _sc_gather_body,
        mesh=plsc.VectorSubcoreMesh(core_axis_name="c", subcore_axis_name="s"),
        out_type=jax.ShapeDtypeStruct((_N, _DP), jnp.float32),
        scratch_types=[
            pltpu.VMEM((_NCHUNK, _CHUNK), jnp.int32),
            pltpu.VMEM((_CHUNK, _DP), jnp.float32),
            pltpu.VMEM((_CHUNK, _DP), jnp.float32),
            pltpu.SemaphoreType.DMA,
        ],
        compiler_params=pltpu.CompilerParams(use_tc_tiling_on_sc=False),
    )
    q_flat = sc_gather(emb_p, idx)

    out4, lsum = pl.pallas_call(
        _combine_body,
        grid=(_B,),
        in_specs=[
            pl.BlockSpec((1, _H, _W, _D), lambda i: (i, 0, 0, 0)),
            pl.BlockSpec((_HW, _DP), lambda i: (i, 0)),
        ],
        out_specs=[
            pl.BlockSpec((1, _D, _H, _W), lambda i: (i, 0, 0, 0)),
            pl.BlockSpec((1, 1), lambda i: (0, 0)),
        ],
        out_shape=[
            jax.ShapeDtypeStruct((_B, _D, _H, _W), jnp.float32),
            jax.ShapeDtypeStruct((1, 1), jnp.float32),
        ],
    )(z, q_flat)

    quantized_indices = idx.reshape(_B, 1, _H, _W)
    m = lsum[0, 0] / jnp.float32(_N * _D)
    loss = m * jnp.float32(0.25) + m * jnp.float32(1.0)
    return (out4, quantized_indices, loss)
